# Initial kernel scaffold; baseline (speedup 1.0000x reference)
import jax
import jax.numpy as jnp
from jax.experimental import pallas as pl


def kernel(pro_x, pro_edge_index, pro_weight, pro_batch, global_rna, local_rna, params):
    raise NotImplementedError("write your pallas kernel here")


# speedup vs baseline: 1.0004x; 1.0004x over previous
import functools
import jax
import jax.numpy as jnp
from jax.experimental import pallas as pl
from jax.experimental.pallas import tpu as pltpu


def _node_mlp_kernel(x_ref, a_ref, w1_ref, b1_ref, w2_ref, b2_ref, s_ref, o_ref):
    eps1 = s_ref[0, 0]
    h = eps1 * x_ref[...] + a_ref[...]
    h = jnp.maximum(h @ w1_ref[...] + b1_ref[...], 0.0)
    h = h @ w2_ref[...] + b2_ref[...]
    h = jnp.maximum(h, 0.0)
    o_ref[...] = h * s_ref[1, :16][None, :] + s_ref[2, :16][None, :]


def _node_mlp(x, aggr, eps, w1, b1, w2, b2, bn_scale, bn_bias):
    n = x.shape[0]
    blk = 2000
    scal = jnp.zeros((3, 128), jnp.float32)
    scal = scal.at[0, 0].set(1.0 + eps)
    scal = scal.at[1, :16].set(bn_scale)
    scal = scal.at[2, :16].set(bn_bias)
    grid = (n // blk,)
    return pl.pallas_call(
        _node_mlp_kernel,
        grid=grid,
        in_specs=[
            pl.BlockSpec((blk, x.shape[1]), lambda i: (i, 0)),
            pl.BlockSpec((blk, x.shape[1]), lambda i: (i, 0)),
            pl.BlockSpec(w1.shape, lambda i: (0, 0)),
            pl.BlockSpec(b1.shape, lambda i: (0,)),
            pl.BlockSpec(w2.shape, lambda i: (0, 0)),
            pl.BlockSpec(b2.shape, lambda i: (0,)),
            pl.BlockSpec((3, 128), lambda i: (0, 0)),
        ],
        out_specs=pl.BlockSpec((blk, 16), lambda i: (i, 0)),
        out_shape=jax.ShapeDtypeStruct((n, 16), jnp.float32),
    )(x, aggr, w1, b1, w2, b2, scal)


def kernel(pro_x, pro_edge_index, pro_weight, pro_batch, global_rna, local_rna, params):
    p = params
    B = global_rna.shape[0]
    N = pro_x.shape[0]

    def conv1d(x, w, b):
        y = jax.lax.conv_general_dilated(x, w, window_strides=(1,), padding='VALID',
                                         dimension_numbers=('NCH', 'OIH', 'NCH'))
        return y + b[None, :, None]

    xrg = p['emb1'][global_rna]
    xrg = conv1d(xrg, p['convr1_w'], p['convr1_b']).reshape(-1, 32 * 121)
    xrg = xrg @ p['fc_xr_w'].T + p['fc_xr_b']
    xrl = p['emb2'][local_rna]
    xrl = conv1d(xrl, p['convr2_w'], p['convr2_b']).reshape(-1, 32 * 121)
    xrl = xrl @ p['fc_xr_w'].T + p['fc_xr_b']
    xc_rna = (xrg + xrl) / 2.0

    ew = pro_weight
    src, dst = pro_edge_index[0], pro_edge_index[1]
    xp = pro_x
    npad = 100000

    for i in range(1, 6):
        istr = '%d' % i
        wv = p['lin%s_w' % istr][:, 0]
        bv = p['lin%s_b' % istr]
        e = ew[:, None] * wv[None, :] + bv[None, :]
        m = jax.nn.relu(xp[src] + e)
        aggr = jax.ops.segment_sum(m, dst, num_segments=N)
        g = p['bn%s_g' % istr]; bb = p['bn%s_b' % istr]
        mu = p['bn%s_m' % istr]; var = p['bn%s_v' % istr]
        inv = g / jnp.sqrt(var + 1e-5)
        bn_scale = inv
        bn_bias = bb - mu * inv
        din = xp.shape[1]
        xp = _node_mlp(xp, aggr, p['eps%s' % istr],
                       p['nn%s_w1' % istr].T, p['nn%s_b1' % istr],
                       p['nn%s_w2' % istr].T, p['nn%s_b2' % istr],
                       bn_scale, bn_bias)

    counts = jax.ops.segment_sum(jnp.ones((N,), jnp.float32), pro_batch, num_segments=B)
    pooled = jax.ops.segment_sum(xp, pro_batch, num_segments=B) / jnp.clip(counts, 1.0)[:, None]
    xp_out = jax.nn.relu(pooled @ p['fc1_xp_w'].T + p['fc1_xp_b'])
    return (xc_rna, xp_out)


# speedup vs baseline: 4.7492x; 4.7474x over previous
import functools
import jax
import jax.numpy as jnp
from jax import lax
from jax.experimental import pallas as pl
from jax.experimental.pallas import tpu as pltpu
from jax.experimental.pallas import tpu_sc as plsc

N_NODES = 100000
N_EDGES = 1600000
NW = 32
CHUNK = 1024
SUB = 128
EPW = 51200
EPAD = EPW * NW
AGG_ROWS = 100096
ROWS_PER_SUB = AGG_ROWS // 16


def _edge_pass_body(x_hbm, srcp, dstp, ewp, wb_hbm, out_hbm,
                    src_v, dst_v, ew_v, rows_v, wb_v, aggr_sh, gsem):
    c = lax.axis_index("c")
    s = lax.axis_index("s")
    wid = s * 2 + c

    pltpu.sync_copy(wb_hbm, wb_v)
    w_v = wb_v[0]
    b_v = wb_v[1]

    def zloop(e, _):
        rows_v[e] = jnp.zeros((16,), jnp.float32)
        return 0
    lax.fori_loop(0, CHUNK, zloop, 0)
    zlo = s * ROWS_PER_SUB
    for q in range(6):
        pltpu.sync_copy(rows_v, aggr_sh.at[pl.ds(zlo + q * 1024, 1024)])
    pltpu.sync_copy(rows_v.at[pl.ds(0, ROWS_PER_SUB - 6144)],
                    aggr_sh.at[pl.ds(zlo + 6144, ROWS_PER_SUB - 6144)])
    plsc.subcore_barrier()

    row0 = wid * (EPW // SUB)

    def chunk_body(t, _):
        rbase = row0 + t * (CHUNK // SUB)
        pltpu.sync_copy(srcp.at[pl.ds(rbase, CHUNK // SUB)], src_v)
        pltpu.sync_copy(dstp.at[pl.ds(rbase, CHUNK // SUB)], dst_v)
        pltpu.sync_copy(ewp.at[pl.ds(rbase * SUB, CHUNK)], ew_v)
        handles = []
        for j in range(CHUNK // SUB):
            handles.append(pltpu.async_copy(
                x_hbm.at[src_v.at[j]],
                rows_v.at[pl.ds(j * SUB, SUB)], gsem))
        for h in handles:
            h.wait()

        def eloop(e, _):
            s16 = plsc.load_gather(ew_v, [jnp.full((16,), e, jnp.int32)])
            m = jnp.maximum(rows_v[e] + s16 * w_v + b_v, 0.0)
            rows_v[e] = m
            return 0
        lax.fori_loop(0, CHUNK, eloop, 0)

        for j in range(CHUNK // SUB):
            pltpu.sync_copy(rows_v.at[pl.ds(j * SUB, SUB)],
                            aggr_sh.at[dst_v.at[j]], add=True)
        return 0

    lax.fori_loop(0, EPW // CHUNK, chunk_body, 0)
    plsc.subcore_barrier()

    pltpu.sync_copy(aggr_sh.at[pl.ds(zlo, ROWS_PER_SUB)],
                    out_hbm.at[c, pl.ds(zlo, ROWS_PER_SUB)])


@jax.jit
def _edge_pass(x16, srcp, dstp, ewp, wb):
    mesh = plsc.VectorSubcoreMesh(core_axis_name="c", subcore_axis_name="s")
    f = pl.kernel(
        _edge_pass_body,
        out_type=jax.ShapeDtypeStruct((2, AGG_ROWS, 16), jnp.float32),
        mesh=mesh,
        scratch_types=[
            pltpu.VMEM((CHUNK // SUB, SUB), jnp.int32),
            pltpu.VMEM((CHUNK // SUB, SUB), jnp.int32),
            pltpu.VMEM((CHUNK,), jnp.float32),
            pltpu.VMEM((CHUNK, 16), jnp.float32),
            pltpu.VMEM((2, 16), jnp.float32),
            pltpu.VMEM_SHARED((AGG_ROWS, 16), jnp.float32),
            pltpu.SemaphoreType.DMA,
        ],
        compiler_params=pltpu.CompilerParams(needs_layout_passes=False,
                                             use_tc_tiling_on_sc=False),
    )
    return f(x16, srcp, dstp, ewp, wb)


def _node_mlp16_kernel(x_ref, a0_ref, a1_ref, w1_ref, b1_ref, w2_ref, b2_ref,
                       s_ref, o_ref):
    eps1 = s_ref[0, 0]
    h = eps1 * x_ref[...] + a0_ref[...] + a1_ref[...]
    h = jnp.maximum(jnp.dot(h, w1_ref[...]) + b1_ref[0], 0.0)
    h = jnp.dot(h, w2_ref[...]) + b2_ref[0]
    h = jnp.maximum(h, 0.0)
    o_ref[...] = h * s_ref[1, :16][None, :] + s_ref[2, :16][None, :]


def _node_mlp16(x, a0, a1, w1, b1, w2, b2, scal):
    n = x.shape[0]
    blk = 4000
    grid = (n // blk,)
    return pl.pallas_call(
        _node_mlp16_kernel,
        grid=grid,
        in_specs=[
            pl.BlockSpec((blk, 16), lambda i: (i, 0)),
            pl.BlockSpec((blk, 16), lambda i: (i, 0)),
            pl.BlockSpec((blk, 16), lambda i: (i, 0)),
            pl.BlockSpec((16, 16), lambda i: (0, 0)),
            pl.BlockSpec((1, 16), lambda i: (0, 0)),
            pl.BlockSpec((16, 16), lambda i: (0, 0)),
            pl.BlockSpec((1, 16), lambda i: (0, 0)),
            pl.BlockSpec((3, 128), lambda i: (0, 0)),
        ],
        out_specs=pl.BlockSpec((blk, 16), lambda i: (i, 0)),
        out_shape=jax.ShapeDtypeStruct((n, 16), jnp.float32),
    )(x, a0, a1, w1, b1, w2, b2, scal)


def _node_mlp48_kernel(x_ref, a0_ref, a1_ref, a2_ref, a3_ref, a4_ref, a5_ref,
                       w1_ref, b1_ref, w2_ref, b2_ref, s_ref, o_ref):
    eps1 = s_ref[0, 0]
    parts = [(a0_ref, a1_ref), (a2_ref, a3_ref), (a4_ref, a5_ref)]
    acc = jnp.zeros((x_ref.shape[0], 16), jnp.float32)
    for fg in range(3):
        p0, p1 = parts[fg]
        hf = eps1 * x_ref[:, fg * 16:(fg + 1) * 16] + p0[...] + p1[...]
        acc = acc + jnp.dot(hf, w1_ref[fg * 16:(fg + 1) * 16, :])
    h = jnp.maximum(acc + b1_ref[0], 0.0)
    h = jnp.dot(h, w2_ref[...]) + b2_ref[0]
    h = jnp.maximum(h, 0.0)
    o_ref[...] = h * s_ref[1, :16][None, :] + s_ref[2, :16][None, :]


def _node_mlp48(x48, aparts, w1p, b1, w2, b2, scal):
    n = x48.shape[0]
    blk = 4000
    grid = (n // blk,)
    specs = [pl.BlockSpec((blk, 48), lambda i: (i, 0))]
    specs += [pl.BlockSpec((blk, 16), lambda i: (i, 0))] * 6
    specs += [
        pl.BlockSpec((48, 16), lambda i: (0, 0)),
        pl.BlockSpec((1, 16), lambda i: (0, 0)),
        pl.BlockSpec((16, 16), lambda i: (0, 0)),
        pl.BlockSpec((1, 16), lambda i: (0, 0)),
        pl.BlockSpec((3, 128), lambda i: (0, 0)),
    ]
    return pl.pallas_call(
        _node_mlp48_kernel,
        grid=grid,
        in_specs=specs,
        out_specs=pl.BlockSpec((blk, 16), lambda i: (i, 0)),
        out_shape=jax.ShapeDtypeStruct((n, 16), jnp.float32),
    )(x48, *aparts, w1p, b1, w2, b2, scal)


def _pool_kernel(batch_ref, x_ref, sum_ref, cnt_ref):
    i = pl.program_id(0)

    @pl.when(i == 0)
    def _():
        sum_ref[...] = jnp.zeros_like(sum_ref)
        cnt_ref[...] = jnp.zeros_like(cnt_ref)

    b = batch_ref[0]
    iota = lax.broadcasted_iota(jnp.int32, (64, b.shape[1]), 0)
    oh = (iota == b).astype(jnp.float32)
    sum_ref[...] += jnp.dot(oh, x_ref[...])
    cnt_ref[...] += jnp.sum(oh, axis=1, keepdims=True)


def _pool(batch2d, xp):
    blk = 4000
    grid = (N_NODES // blk,)
    return pl.pallas_call(
        _pool_kernel,
        grid=grid,
        in_specs=[
            pl.BlockSpec((1, 1, blk), lambda i: (i, 0, 0)),
            pl.BlockSpec((blk, 16), lambda i: (i, 0)),
        ],
        out_specs=[
            pl.BlockSpec((64, 16), lambda i: (0, 0)),
            pl.BlockSpec((64, 1), lambda i: (0, 0)),
        ],
        out_shape=[
            jax.ShapeDtypeStruct((64, 16), jnp.float32),
            jax.ShapeDtypeStruct((64, 1), jnp.float32),
        ],
    )(batch2d, xp)


def _fc_pool_kernel(s_ref, c_ref, w_ref, b_ref, o_ref):
    pooled = s_ref[...] / jnp.maximum(c_ref[...], 1.0)
    o_ref[...] = jnp.maximum(jnp.dot(pooled, w_ref[...]) + b_ref[0], 0.0)


def _fc_pool(psum, pcnt, w, b):
    return pl.pallas_call(
        _fc_pool_kernel,
        in_specs=[
            pl.BlockSpec((64, 16), lambda: (0, 0)),
            pl.BlockSpec((64, 1), lambda: (0, 0)),
            pl.BlockSpec((16, 128), lambda: (0, 0)),
            pl.BlockSpec((1, 128), lambda: (0, 0)),
        ],
        out_specs=pl.BlockSpec((64, 128), lambda: (0, 0)),
        out_shape=jax.ShapeDtypeStruct((64, 128), jnp.float32),
    )(psum, pcnt, w, b)


def kernel(pro_x, pro_edge_index, pro_weight, pro_batch, global_rna, local_rna, params):
    p = params
    B = global_rna.shape[0]
    N = pro_x.shape[0]

    def conv1d(x, w, b):
        y = jax.lax.conv_general_dilated(x, w, window_strides=(1,), padding='VALID',
                                         dimension_numbers=('NCH', 'OIH', 'NCH'))
        return y + b[None, :, None]

    xrg = p['emb1'][global_rna]
    xrg = conv1d(xrg, p['convr1_w'], p['convr1_b']).reshape(-1, 32 * 121)
    xrg = xrg @ p['fc_xr_w'].T + p['fc_xr_b']
    xrl = p['emb2'][local_rna]
    xrl = conv1d(xrl, p['convr2_w'], p['convr2_b']).reshape(-1, 32 * 121)
    xrl = xrl @ p['fc_xr_w'].T + p['fc_xr_b']
    xc_rna = (xrg + xrl) / 2.0

    src = pro_edge_index[0]
    dst = pro_edge_index[1]
    ew = pro_weight
    pad = EPAD - N_EDGES
    srcp = jnp.concatenate([src, jnp.zeros((pad,), jnp.int32)]).reshape(EPAD // SUB, SUB)
    dstp = jnp.concatenate([dst, jnp.full((pad,), AGG_ROWS - 1, jnp.int32)]).reshape(EPAD // SUB, SUB)
    ewp = jnp.concatenate([ew, jnp.zeros((pad,), jnp.float32)])

    def bn_fold(i):
        istr = '%d' % i
        g = p['bn%s_g' % istr]; bb = p['bn%s_b' % istr]
        mu = p['bn%s_m' % istr]; var = p['bn%s_v' % istr]
        inv = g / jnp.sqrt(var + 1e-5)
        return inv, bb - mu * inv

    def make_scal(i, bn_scale, bn_bias):
        scal = jnp.zeros((3, 128), jnp.float32)
        scal = scal.at[0, 0].set(1.0 + p['eps%d' % i])
        scal = scal.at[1, :16].set(bn_scale)
        scal = scal.at[2, :16].set(bn_bias)
        return scal

    x0p = jnp.pad(pro_x, ((0, 0), (0, 15)))
    w1v = jnp.pad(p['lin1_w'][:, 0], (0, 15))
    b1v = jnp.pad(p['lin1_b'], (0, 15))
    aparts = []
    for fg in range(3):
        x16 = x0p[:, fg * 16:(fg + 1) * 16]
        wb = jnp.stack([w1v[fg * 16:(fg + 1) * 16], b1v[fg * 16:(fg + 1) * 16]])
        part = _edge_pass(x16, srcp, dstp, ewp, wb)
        aparts.append(part[0, :N_NODES])
        aparts.append(part[1, :N_NODES])
    bn_s, bn_b = bn_fold(1)
    scal = make_scal(1, bn_s, bn_b)
    w1p = jnp.pad(p['nn1_w1'].T, ((0, 15), (0, 0)))
    xp = _node_mlp48(x0p, aparts, w1p, p['nn1_b1'][None, :],
                     p['nn1_w2'].T, p['nn1_b2'][None, :], scal)

    for i in range(2, 6):
        istr = '%d' % i
        wb = jnp.stack([p['lin%s_w' % istr][:, 0], p['lin%s_b' % istr]])
        part = _edge_pass(xp, srcp, dstp, ewp, wb)
        bn_s, bn_b = bn_fold(i)
        scal = make_scal(i, bn_s, bn_b)
        xp = _node_mlp16(xp, part[0, :N_NODES], part[1, :N_NODES],
                         p['nn%s_w1' % istr].T, p['nn%s_b1' % istr][None, :],
                         p['nn%s_w2' % istr].T, p['nn%s_b2' % istr][None, :],
                         scal)

    batch2d = pro_batch.reshape(N_NODES // 4000, 1, 4000)
    psum, pcnt = _pool(batch2d, xp)
    xp_out = _fc_pool(psum, pcnt, p['fc1_xp_w'].T, p['fc1_xp_b'][None, :])
    return (xc_rna, xp_out)


# speedup vs baseline: 6.6920x; 1.4091x over previous
import functools
import jax
import jax.numpy as jnp
from jax import lax
from jax.experimental import pallas as pl
from jax.experimental.pallas import tpu as pltpu
from jax.experimental.pallas import tpu_sc as plsc

N_NODES = 100000
N_EDGES = 1600000
NW = 32
CHUNK = 1024
SUB = 128
EPW = 51200
EPAD = EPW * NW
AGG_ROWS = 100096
ROWS_PER_SUB = AGG_ROWS // 16


def _edge_pass_body(x_hbm, srcp, dstp, ewp, wb_hbm, out_hbm,
                    src_v, dst_v, ew_v, rows_v, wb_v, aggr_sh, gsem):
    c = lax.axis_index("c")
    s = lax.axis_index("s")
    wid = s * 2 + c

    pltpu.sync_copy(wb_hbm, wb_v)
    w_v = wb_v[0]
    b_v = wb_v[1]

    def zloop(e, _):
        rows_v[e] = jnp.zeros((16,), jnp.float32)
        return 0
    lax.fori_loop(0, CHUNK, zloop, 0)
    zlo = s * ROWS_PER_SUB
    for q in range(6):
        pltpu.sync_copy(rows_v, aggr_sh.at[pl.ds(zlo + q * 1024, 1024)])
    pltpu.sync_copy(rows_v.at[pl.ds(0, ROWS_PER_SUB - 6144)],
                    aggr_sh.at[pl.ds(zlo + 6144, ROWS_PER_SUB - 6144)])
    plsc.subcore_barrier()

    row0 = wid * (EPW // SUB)

    def chunk_body(t, _):
        rbase = row0 + t * (CHUNK // SUB)
        pltpu.sync_copy(srcp.at[pl.ds(rbase, CHUNK // SUB)], src_v)
        pltpu.sync_copy(dstp.at[pl.ds(rbase, CHUNK // SUB)], dst_v)
        pltpu.sync_copy(ewp.at[pl.ds(rbase * SUB, CHUNK)], ew_v)
        handles = []
        for j in range(CHUNK // SUB):
            handles.append(pltpu.async_copy(
                x_hbm.at[src_v.at[j]],
                rows_v.at[pl.ds(j * SUB, SUB)], gsem))
        for h in handles:
            h.wait()

        @plsc.parallel_loop(0, CHUNK, unroll=8)
        def _(e):
            s16 = plsc.load_gather(ew_v, [jnp.full((16,), e, jnp.int32)])
            m = jnp.maximum(rows_v[e] + s16 * w_v + b_v, 0.0)
            rows_v[e] = m

        for j in range(CHUNK // SUB):
            pltpu.sync_copy(rows_v.at[pl.ds(j * SUB, SUB)],
                            aggr_sh.at[dst_v.at[j]], add=True)
        return 0

    lax.fori_loop(0, EPW // CHUNK, chunk_body, 0)
    plsc.subcore_barrier()

    pltpu.sync_copy(aggr_sh.at[pl.ds(zlo, ROWS_PER_SUB)],
                    out_hbm.at[c, pl.ds(zlo, ROWS_PER_SUB)])


@jax.jit
def _edge_pass(x16, srcp, dstp, ewp, wb):
    mesh = plsc.VectorSubcoreMesh(core_axis_name="c", subcore_axis_name="s")
    f = pl.kernel(
        _edge_pass_body,
        out_type=jax.ShapeDtypeStruct((2, AGG_ROWS, 16), jnp.float32),
        mesh=mesh,
        scratch_types=[
            pltpu.VMEM((CHUNK // SUB, SUB), jnp.int32),
            pltpu.VMEM((CHUNK // SUB, SUB), jnp.int32),
            pltpu.VMEM((CHUNK,), jnp.float32),
            pltpu.VMEM((CHUNK, 16), jnp.float32),
            pltpu.VMEM((2, 16), jnp.float32),
            pltpu.VMEM_SHARED((AGG_ROWS, 16), jnp.float32),
            pltpu.SemaphoreType.DMA,
        ],
        compiler_params=pltpu.CompilerParams(needs_layout_passes=False,
                                             use_tc_tiling_on_sc=False),
    )
    return f(x16, srcp, dstp, ewp, wb)


def _node_mlp16_kernel(x_ref, a0_ref, a1_ref, w1_ref, b1_ref, w2_ref, b2_ref,
                       s_ref, o_ref):
    eps1 = s_ref[0, 0]
    h = eps1 * x_ref[...] + a0_ref[...] + a1_ref[...]
    h = jnp.maximum(jnp.dot(h, w1_ref[...]) + b1_ref[0], 0.0)
    h = jnp.dot(h, w2_ref[...]) + b2_ref[0]
    h = jnp.maximum(h, 0.0)
    o_ref[...] = h * s_ref[1, :16][None, :] + s_ref[2, :16][None, :]


def _node_mlp16(x, a0, a1, w1, b1, w2, b2, scal):
    n = x.shape[0]
    blk = 4000
    grid = (n // blk,)
    return pl.pallas_call(
        _node_mlp16_kernel,
        grid=grid,
        in_specs=[
            pl.BlockSpec((blk, 16), lambda i: (i, 0)),
            pl.BlockSpec((blk, 16), lambda i: (i, 0)),
            pl.BlockSpec((blk, 16), lambda i: (i, 0)),
            pl.BlockSpec((16, 16), lambda i: (0, 0)),
            pl.BlockSpec((1, 16), lambda i: (0, 0)),
            pl.BlockSpec((16, 16), lambda i: (0, 0)),
            pl.BlockSpec((1, 16), lambda i: (0, 0)),
            pl.BlockSpec((3, 128), lambda i: (0, 0)),
        ],
        out_specs=pl.BlockSpec((blk, 16), lambda i: (i, 0)),
        out_shape=jax.ShapeDtypeStruct((n, 16), jnp.float32),
    )(x, a0, a1, w1, b1, w2, b2, scal)


def _node_mlp48_kernel(x_ref, a0_ref, a1_ref, a2_ref, a3_ref, a4_ref, a5_ref,
                       w1_ref, b1_ref, w2_ref, b2_ref, s_ref, o_ref):
    eps1 = s_ref[0, 0]
    parts = [(a0_ref, a1_ref), (a2_ref, a3_ref), (a4_ref, a5_ref)]
    acc = jnp.zeros((x_ref.shape[0], 16), jnp.float32)
    for fg in range(3):
        p0, p1 = parts[fg]
        hf = eps1 * x_ref[:, fg * 16:(fg + 1) * 16] + p0[...] + p1[...]
        acc = acc + jnp.dot(hf, w1_ref[fg * 16:(fg + 1) * 16, :])
    h = jnp.maximum(acc + b1_ref[0], 0.0)
    h = jnp.dot(h, w2_ref[...]) + b2_ref[0]
    h = jnp.maximum(h, 0.0)
    o_ref[...] = h * s_ref[1, :16][None, :] + s_ref[2, :16][None, :]


def _node_mlp48(x48, aparts, w1p, b1, w2, b2, scal):
    n = x48.shape[0]
    blk = 4000
    grid = (n // blk,)
    specs = [pl.BlockSpec((blk, 48), lambda i: (i, 0))]
    specs += [pl.BlockSpec((blk, 16), lambda i: (i, 0))] * 6
    specs += [
        pl.BlockSpec((48, 16), lambda i: (0, 0)),
        pl.BlockSpec((1, 16), lambda i: (0, 0)),
        pl.BlockSpec((16, 16), lambda i: (0, 0)),
        pl.BlockSpec((1, 16), lambda i: (0, 0)),
        pl.BlockSpec((3, 128), lambda i: (0, 0)),
    ]
    return pl.pallas_call(
        _node_mlp48_kernel,
        grid=grid,
        in_specs=specs,
        out_specs=pl.BlockSpec((blk, 16), lambda i: (i, 0)),
        out_shape=jax.ShapeDtypeStruct((n, 16), jnp.float32),
    )(x48, *aparts, w1p, b1, w2, b2, scal)


def _pool_kernel(batch_ref, x_ref, sum_ref, cnt_ref):
    i = pl.program_id(0)

    @pl.when(i == 0)
    def _():
        sum_ref[...] = jnp.zeros_like(sum_ref)
        cnt_ref[...] = jnp.zeros_like(cnt_ref)

    b = batch_ref[0]
    iota = lax.broadcasted_iota(jnp.int32, (64, b.shape[1]), 0)
    oh = (iota == b).astype(jnp.float32)
    sum_ref[...] += jnp.dot(oh, x_ref[...])
    cnt_ref[...] += jnp.sum(oh, axis=1, keepdims=True)


def _pool(batch2d, xp):
    blk = 4000
    grid = (N_NODES // blk,)
    return pl.pallas_call(
        _pool_kernel,
        grid=grid,
        in_specs=[
            pl.BlockSpec((1, 1, blk), lambda i: (i, 0, 0)),
            pl.BlockSpec((blk, 16), lambda i: (i, 0)),
        ],
        out_specs=[
            pl.BlockSpec((64, 16), lambda i: (0, 0)),
            pl.BlockSpec((64, 1), lambda i: (0, 0)),
        ],
        out_shape=[
            jax.ShapeDtypeStruct((64, 16), jnp.float32),
            jax.ShapeDtypeStruct((64, 1), jnp.float32),
        ],
    )(batch2d, xp)


def _fc_pool_kernel(s_ref, c_ref, w_ref, b_ref, o_ref):
    pooled = s_ref[...] / jnp.maximum(c_ref[...], 1.0)
    o_ref[...] = jnp.maximum(jnp.dot(pooled, w_ref[...]) + b_ref[0], 0.0)


def _fc_pool(psum, pcnt, w, b):
    return pl.pallas_call(
        _fc_pool_kernel,
        in_specs=[
            pl.BlockSpec((64, 16), lambda: (0, 0)),
            pl.BlockSpec((64, 1), lambda: (0, 0)),
            pl.BlockSpec((16, 128), lambda: (0, 0)),
            pl.BlockSpec((1, 128), lambda: (0, 0)),
        ],
        out_specs=pl.BlockSpec((64, 128), lambda: (0, 0)),
        out_shape=jax.ShapeDtypeStruct((64, 128), jnp.float32),
    )(psum, pcnt, w, b)


def kernel(pro_x, pro_edge_index, pro_weight, pro_batch, global_rna, local_rna, params):
    p = params
    B = global_rna.shape[0]
    N = pro_x.shape[0]

    def conv1d(x, w, b):
        y = jax.lax.conv_general_dilated(x, w, window_strides=(1,), padding='VALID',
                                         dimension_numbers=('NCH', 'OIH', 'NCH'))
        return y + b[None, :, None]

    xrg = p['emb1'][global_rna]
    xrg = conv1d(xrg, p['convr1_w'], p['convr1_b']).reshape(-1, 32 * 121)
    xrg = xrg @ p['fc_xr_w'].T + p['fc_xr_b']
    xrl = p['emb2'][local_rna]
    xrl = conv1d(xrl, p['convr2_w'], p['convr2_b']).reshape(-1, 32 * 121)
    xrl = xrl @ p['fc_xr_w'].T + p['fc_xr_b']
    xc_rna = (xrg + xrl) / 2.0

    src = pro_edge_index[0]
    dst = pro_edge_index[1]
    ew = pro_weight
    pad = EPAD - N_EDGES
    srcp = jnp.concatenate([src, jnp.zeros((pad,), jnp.int32)]).reshape(EPAD // SUB, SUB)
    dstp = jnp.concatenate([dst, jnp.full((pad,), AGG_ROWS - 1, jnp.int32)]).reshape(EPAD // SUB, SUB)
    ewp = jnp.concatenate([ew, jnp.zeros((pad,), jnp.float32)])

    def bn_fold(i):
        istr = '%d' % i
        g = p['bn%s_g' % istr]; bb = p['bn%s_b' % istr]
        mu = p['bn%s_m' % istr]; var = p['bn%s_v' % istr]
        inv = g / jnp.sqrt(var + 1e-5)
        return inv, bb - mu * inv

    def make_scal(i, bn_scale, bn_bias):
        scal = jnp.zeros((3, 128), jnp.float32)
        scal = scal.at[0, 0].set(1.0 + p['eps%d' % i])
        scal = scal.at[1, :16].set(bn_scale)
        scal = scal.at[2, :16].set(bn_bias)
        return scal

    x0p = jnp.pad(pro_x, ((0, 0), (0, 15)))
    w1v = jnp.pad(p['lin1_w'][:, 0], (0, 15))
    b1v = jnp.pad(p['lin1_b'], (0, 15))
    aparts = []
    for fg in range(3):
        x16 = x0p[:, fg * 16:(fg + 1) * 16]
        wb = jnp.stack([w1v[fg * 16:(fg + 1) * 16], b1v[fg * 16:(fg + 1) * 16]])
        part = _edge_pass(x16, srcp, dstp, ewp, wb)
        aparts.append(part[0, :N_NODES])
        aparts.append(part[1, :N_NODES])
    bn_s, bn_b = bn_fold(1)
    scal = make_scal(1, bn_s, bn_b)
    w1p = jnp.pad(p['nn1_w1'].T, ((0, 15), (0, 0)))
    xp = _node_mlp48(x0p, aparts, w1p, p['nn1_b1'][None, :],
                     p['nn1_w2'].T, p['nn1_b2'][None, :], scal)

    for i in range(2, 6):
        istr = '%d' % i
        wb = jnp.stack([p['lin%s_w' % istr][:, 0], p['lin%s_b' % istr]])
        part = _edge_pass(xp, srcp, dstp, ewp, wb)
        bn_s, bn_b = bn_fold(i)
        scal = make_scal(i, bn_s, bn_b)
        xp = _node_mlp16(xp, part[0, :N_NODES], part[1, :N_NODES],
                         p['nn%s_w1' % istr].T, p['nn%s_b1' % istr][None, :],
                         p['nn%s_w2' % istr].T, p['nn%s_b2' % istr][None, :],
                         scal)

    batch2d = pro_batch.reshape(N_NODES // 4000, 1, 4000)
    psum, pcnt = _pool(batch2d, xp)
    xp_out = _fc_pool(psum, pcnt, p['fc1_xp_w'].T, p['fc1_xp_b'][None, :])
    return (xc_rna, xp_out)


# speedup vs baseline: 8.7662x; 1.3099x over previous
import functools
import jax
import jax.numpy as jnp
from jax import lax
from jax.experimental import pallas as pl
from jax.experimental.pallas import tpu as pltpu
from jax.experimental.pallas import tpu_sc as plsc

N_NODES = 100000
N_EDGES = 1600000
NW = 32
CHUNK = 512
SUB = 128
EPW = 51200
EPAD = EPW * NW
AGG_ROWS = 100096
ROWS_PER_SUB = AGG_ROWS // 16


def _edge_pass_body(x_hbm, sdp, ewp, wb_hbm, out_hbm,
                    sd_v, ew_v0, ew_v1, rows_v, wb_v, aggr_sh, gsem0,
                    gsem1, ssem0, ssem1):
    c = lax.axis_index("c")
    s = lax.axis_index("s")
    wid = s * 2 + c
    NCH = EPW // CHUNK
    NSUB = CHUNK // SUB

    pltpu.sync_copy(wb_hbm, wb_v)
    w_v = wb_v[0]
    b_v = wb_v[1]

    @plsc.parallel_loop(0, CHUNK, unroll=8)
    def _(e):
        rows_v[e] = jnp.zeros((16,), jnp.float32)
    zlo = s * ROWS_PER_SUB
    nfull = ROWS_PER_SUB // CHUNK
    for q in range(nfull):
        pltpu.sync_copy(rows_v.at[pl.ds(0, CHUNK)],
                        aggr_sh.at[pl.ds(zlo + q * CHUNK, CHUNK)])
    rem = ROWS_PER_SUB - nfull * CHUNK
    if rem:
        pltpu.sync_copy(rows_v.at[pl.ds(0, rem)],
                        aggr_sh.at[pl.ds(zlo + nfull * CHUNK, rem)])
    plsc.subcore_barrier()

    row0 = wid * (EPW // SUB)
    ews = (ew_v0, ew_v1)
    gsems = (gsem0, gsem1)
    ssems = (ssem0, ssem1)

    def fetch(t, pb):
        rbase = row0 + t * NSUB
        pltpu.sync_copy(sdp.at[pl.ds(rbase, NSUB)],
                        sd_v.at[pl.ds(pb * NSUB, NSUB)])
        pltpu.sync_copy(ewp.at[pl.ds(rbase * SUB, CHUNK)], ews[pb])
        for j in range(NSUB):
            pltpu.async_copy(
                x_hbm.at[sd_v.at[pb * NSUB + j, 0]],
                rows_v.at[pl.ds(pb * CHUNK + j * SUB, SUB)], gsems[pb])

    def drain_gather(pb):
        pltpu.make_async_copy(
            x_hbm.at[pl.ds(0, CHUNK)],
            rows_v.at[pl.ds(pb * CHUNK, CHUNK)], gsems[pb]).wait()

    def drain_scatter(pb):
        pltpu.make_async_copy(
            x_hbm.at[pl.ds(0, CHUNK)],
            rows_v.at[pl.ds(pb * CHUNK, CHUNK)], ssems[pb]).wait()

    fetch(0, 0)

    def chunk_body(t, _):
        pb0 = lax.rem(t, 2)

        @pl.when(t + 1 < NCH)
        def _():
            @pl.when(t >= 1)
            def _():
                for pb in range(2):
                    @pl.when(pb0 != pb)
                    def _():
                        drain_scatter(pb)
            for pb in range(2):
                @pl.when(pb0 != pb)
                def _():
                    fetch(t + 1, pb)

        for pb in range(2):
            @pl.when(pb0 == pb)
            def _():
                drain_gather(pb)

                @plsc.parallel_loop(0, CHUNK, unroll=8)
                def _(e):
                    s16 = plsc.load_gather(ews[pb], [jnp.full((16,), e, jnp.int32)])
                    m = jnp.maximum(rows_v[pb * CHUNK + e] + s16 * w_v + b_v, 0.0)
                    rows_v[pb * CHUNK + e] = m

                for j in range(NSUB):
                    pltpu.async_copy(
                        rows_v.at[pl.ds(pb * CHUNK + j * SUB, SUB)],
                        aggr_sh.at[sd_v.at[pb * NSUB + j, 1]], ssems[pb],
                        add=True)
        return 0

    lax.fori_loop(0, NCH, chunk_body, 0)
    drain_scatter((NCH - 2) % 2)
    drain_scatter((NCH - 1) % 2)
    plsc.subcore_barrier()

    pltpu.sync_copy(aggr_sh.at[pl.ds(zlo, ROWS_PER_SUB)],
                    out_hbm.at[c, pl.ds(zlo, ROWS_PER_SUB)])


@jax.jit
def _edge_pass(x16, sdp, ewp, wb):
    mesh = plsc.VectorSubcoreMesh(core_axis_name="c", subcore_axis_name="s")
    f = pl.kernel(
        _edge_pass_body,
        out_type=jax.ShapeDtypeStruct((2, AGG_ROWS, 16), jnp.float32),
        mesh=mesh,
        scratch_types=[
            pltpu.VMEM((2 * (CHUNK // SUB), 2, SUB), jnp.int32),
            pltpu.VMEM((CHUNK,), jnp.float32),
            pltpu.VMEM((CHUNK,), jnp.float32),
            pltpu.VMEM((2 * CHUNK, 16), jnp.float32),
            pltpu.VMEM((2, 16), jnp.float32),
            pltpu.VMEM_SHARED((AGG_ROWS, 16), jnp.float32),
            pltpu.SemaphoreType.DMA,
            pltpu.SemaphoreType.DMA,
            pltpu.SemaphoreType.DMA,
            pltpu.SemaphoreType.DMA,
        ],
        compiler_params=pltpu.CompilerParams(needs_layout_passes=False,
                                             use_tc_tiling_on_sc=False),
    )
    return f(x16, sdp, ewp, wb)


def _node_mlp16_kernel(x_ref, a0_ref, a1_ref, w1_ref, b1_ref, w2_ref, b2_ref,
                       s_ref, o_ref):
    eps1 = s_ref[0, 0]
    h = eps1 * x_ref[...] + a0_ref[...] + a1_ref[...]
    h = jnp.maximum(jnp.dot(h, w1_ref[...]) + b1_ref[0], 0.0)
    h = jnp.dot(h, w2_ref[...]) + b2_ref[0]
    h = jnp.maximum(h, 0.0)
    o_ref[...] = h * s_ref[1, :16][None, :] + s_ref[2, :16][None, :]


def _node_mlp16(x, a0, a1, w1, b1, w2, b2, scal):
    n = x.shape[0]
    blk = 4000
    grid = (n // blk,)
    return pl.pallas_call(
        _node_mlp16_kernel,
        grid=grid,
        in_specs=[
            pl.BlockSpec((blk, 16), lambda i: (i, 0)),
            pl.BlockSpec((blk, 16), lambda i: (i, 0)),
            pl.BlockSpec((blk, 16), lambda i: (i, 0)),
            pl.BlockSpec((16, 16), lambda i: (0, 0)),
            pl.BlockSpec((1, 16), lambda i: (0, 0)),
            pl.BlockSpec((16, 16), lambda i: (0, 0)),
            pl.BlockSpec((1, 16), lambda i: (0, 0)),
            pl.BlockSpec((3, 128), lambda i: (0, 0)),
        ],
        out_specs=pl.BlockSpec((blk, 16), lambda i: (i, 0)),
        out_shape=jax.ShapeDtypeStruct((n, 16), jnp.float32),
    )(x, a0, a1, w1, b1, w2, b2, scal)


def _node_mlp48_kernel(x_ref, a0_ref, a1_ref, a2_ref, a3_ref, a4_ref, a5_ref,
                       w1_ref, b1_ref, w2_ref, b2_ref, s_ref, o_ref):
    eps1 = s_ref[0, 0]
    parts = [(a0_ref, a1_ref), (a2_ref, a3_ref), (a4_ref, a5_ref)]
    acc = jnp.zeros((x_ref.shape[0], 16), jnp.float32)
    for fg in range(3):
        p0, p1 = parts[fg]
        hf = eps1 * x_ref[:, fg * 16:(fg + 1) * 16] + p0[...] + p1[...]
        acc = acc + jnp.dot(hf, w1_ref[fg * 16:(fg + 1) * 16, :])
    h = jnp.maximum(acc + b1_ref[0], 0.0)
    h = jnp.dot(h, w2_ref[...]) + b2_ref[0]
    h = jnp.maximum(h, 0.0)
    o_ref[...] = h * s_ref[1, :16][None, :] + s_ref[2, :16][None, :]


def _node_mlp48(x48, aparts, w1p, b1, w2, b2, scal):
    n = x48.shape[0]
    blk = 4000
    grid = (n // blk,)
    specs = [pl.BlockSpec((blk, 48), lambda i: (i, 0))]
    specs += [pl.BlockSpec((blk, 16), lambda i: (i, 0))] * 6
    specs += [
        pl.BlockSpec((48, 16), lambda i: (0, 0)),
        pl.BlockSpec((1, 16), lambda i: (0, 0)),
        pl.BlockSpec((16, 16), lambda i: (0, 0)),
        pl.BlockSpec((1, 16), lambda i: (0, 0)),
        pl.BlockSpec((3, 128), lambda i: (0, 0)),
    ]
    return pl.pallas_call(
        _node_mlp48_kernel,
        grid=grid,
        in_specs=specs,
        out_specs=pl.BlockSpec((blk, 16), lambda i: (i, 0)),
        out_shape=jax.ShapeDtypeStruct((n, 16), jnp.float32),
    )(x48, *aparts, w1p, b1, w2, b2, scal)


def _rna_conv_kernel(ids_ref, emb_ref, w2d_ref, b_ref, o_ref):
    L = ids_ref.shape[1]
    ids = ids_ref[0]
    iota = lax.broadcasted_iota(jnp.int32, (L, 128), 1)
    oh = (iota == ids).astype(jnp.float32)
    xg = jnp.dot(oh, emb_ref[...])
    zz = jnp.dot(w2d_ref[...], xg)
    y = b_ref[...]
    acc = jnp.zeros((32, 121), jnp.float32) + y
    for k in range(8):
        acc = acc + zz[k * 32:(k + 1) * 32, k:k + 121]
    o_ref[0] = acc


def _rna_conv(ids3d, embp, w2d, b):
    Bn, L, _ = ids3d.shape
    return pl.pallas_call(
        _rna_conv_kernel,
        grid=(Bn,),
        in_specs=[
            pl.BlockSpec((1, L, 1), lambda i: (i, 0, 0)),
            pl.BlockSpec((128, 128), lambda i: (0, 0)),
            pl.BlockSpec((256, L), lambda i: (0, 0)),
            pl.BlockSpec((32, 1), lambda i: (0, 0)),
        ],
        out_specs=pl.BlockSpec((1, 32, 121), lambda i: (i, 0, 0)),
        out_shape=jax.ShapeDtypeStruct((Bn, 32, 121), jnp.float32),
    )(ids3d, embp, w2d, b)


def _rna_fc_kernel(yg_ref, yl_ref, w_ref, b_ref, o_ref):
    s = (yg_ref[...] + yl_ref[...]) * 0.5
    o_ref[...] = jnp.dot(s, w_ref[...]) + b_ref[...]


def _rna_fc(yg, yl, w, b):
    Bn, K = yg.shape
    return pl.pallas_call(
        _rna_fc_kernel,
        in_specs=[
            pl.BlockSpec((Bn, K), lambda: (0, 0)),
            pl.BlockSpec((Bn, K), lambda: (0, 0)),
            pl.BlockSpec((K, 128), lambda: (0, 0)),
            pl.BlockSpec((1, 128), lambda: (0, 0)),
        ],
        out_specs=pl.BlockSpec((Bn, 128), lambda: (0, 0)),
        out_shape=jax.ShapeDtypeStruct((Bn, 128), jnp.float32),
    )(yg, yl, w, b)


def _pool_kernel(batch_ref, x_ref, sum_ref, cnt_ref):
    i = pl.program_id(0)

    @pl.when(i == 0)
    def _():
        sum_ref[...] = jnp.zeros_like(sum_ref)
        cnt_ref[...] = jnp.zeros_like(cnt_ref)

    b = batch_ref[0]
    iota = lax.broadcasted_iota(jnp.int32, (64, b.shape[1]), 0)
    oh = (iota == b).astype(jnp.float32)
    sum_ref[...] += jnp.dot(oh, x_ref[...])
    cnt_ref[...] += jnp.sum(oh, axis=1, keepdims=True)


def _pool(batch2d, xp):
    blk = 4000
    grid = (N_NODES // blk,)
    return pl.pallas_call(
        _pool_kernel,
        grid=grid,
        in_specs=[
            pl.BlockSpec((1, 1, blk), lambda i: (i, 0, 0)),
            pl.BlockSpec((blk, 16), lambda i: (i, 0)),
        ],
        out_specs=[
            pl.BlockSpec((64, 16), lambda i: (0, 0)),
            pl.BlockSpec((64, 1), lambda i: (0, 0)),
        ],
        out_shape=[
            jax.ShapeDtypeStruct((64, 16), jnp.float32),
            jax.ShapeDtypeStruct((64, 1), jnp.float32),
        ],
    )(batch2d, xp)


def _fc_pool_kernel(s_ref, c_ref, w_ref, b_ref, o_ref):
    pooled = s_ref[...] / jnp.maximum(c_ref[...], 1.0)
    o_ref[...] = jnp.maximum(jnp.dot(pooled, w_ref[...]) + b_ref[0], 0.0)


def _fc_pool(psum, pcnt, w, b):
    return pl.pallas_call(
        _fc_pool_kernel,
        in_specs=[
            pl.BlockSpec((64, 16), lambda: (0, 0)),
            pl.BlockSpec((64, 1), lambda: (0, 0)),
            pl.BlockSpec((16, 128), lambda: (0, 0)),
            pl.BlockSpec((1, 128), lambda: (0, 0)),
        ],
        out_specs=pl.BlockSpec((64, 128), lambda: (0, 0)),
        out_shape=jax.ShapeDtypeStruct((64, 128), jnp.float32),
    )(psum, pcnt, w, b)


def kernel(pro_x, pro_edge_index, pro_weight, pro_batch, global_rna, local_rna, params):
    p = params
    B = global_rna.shape[0]
    N = pro_x.shape[0]

    def conv1d(x, w, b):
        y = jax.lax.conv_general_dilated(x, w, window_strides=(1,), padding='VALID',
                                         dimension_numbers=('NCH', 'OIH', 'NCH'))
        return y + b[None, :, None]

    emb1p = jnp.zeros((128, 128), jnp.float32).at[:5].set(p['emb1'])
    emb2p = jnp.zeros((128, 128), jnp.float32).at[:65].set(p['emb2'])
    w1_2d = jnp.transpose(p['convr1_w'], (2, 0, 1)).reshape(256, 3000)
    w2_2d = jnp.transpose(p['convr2_w'], (2, 0, 1)).reshape(256, 2998)
    yg = _rna_conv(global_rna.reshape(B, 3000, 1), emb1p, w1_2d,
                   p['convr1_b'].reshape(32, 1))
    yl = _rna_conv(local_rna.reshape(B, 2998, 1), emb2p, w2_2d,
                   p['convr2_b'].reshape(32, 1))
    xc_rna = _rna_fc(yg.reshape(B, 32 * 121), yl.reshape(B, 32 * 121),
                     p['fc_xr_w'].T, p['fc_xr_b'].reshape(1, 128))

    src = pro_edge_index[0]
    dst = pro_edge_index[1]
    ew = pro_weight
    pad = EPAD - N_EDGES
    srcp = jnp.concatenate([src, jnp.zeros((pad,), jnp.int32)]).reshape(EPAD // SUB, SUB)
    dstp = jnp.concatenate([dst, jnp.full((pad,), AGG_ROWS - 1, jnp.int32)]).reshape(EPAD // SUB, SUB)
    sdp = jnp.stack([srcp, dstp], axis=1)
    ewp = jnp.concatenate([ew, jnp.zeros((pad,), jnp.float32)])

    def bn_fold(i):
        istr = '%d' % i
        g = p['bn%s_g' % istr]; bb = p['bn%s_b' % istr]
        mu = p['bn%s_m' % istr]; var = p['bn%s_v' % istr]
        inv = g / jnp.sqrt(var + 1e-5)
        return inv, bb - mu * inv

    def make_scal(i, bn_scale, bn_bias):
        scal = jnp.zeros((3, 128), jnp.float32)
        scal = scal.at[0, 0].set(1.0 + p['eps%d' % i])
        scal = scal.at[1, :16].set(bn_scale)
        scal = scal.at[2, :16].set(bn_bias)
        return scal

    x0p = jnp.pad(pro_x, ((0, 0), (0, 15)))
    w1v = jnp.pad(p['lin1_w'][:, 0], (0, 15))
    b1v = jnp.pad(p['lin1_b'], (0, 15))
    aparts = []
    for fg in range(3):
        x16 = x0p[:, fg * 16:(fg + 1) * 16]
        wb = jnp.stack([w1v[fg * 16:(fg + 1) * 16], b1v[fg * 16:(fg + 1) * 16]])
        part = _edge_pass(x16, sdp, ewp, wb)
        aparts.append(part[0, :N_NODES])
        aparts.append(part[1, :N_NODES])
    bn_s, bn_b = bn_fold(1)
    scal = make_scal(1, bn_s, bn_b)
    w1p = jnp.pad(p['nn1_w1'].T, ((0, 15), (0, 0)))
    xp = _node_mlp48(x0p, aparts, w1p, p['nn1_b1'][None, :],
                     p['nn1_w2'].T, p['nn1_b2'][None, :], scal)

    for i in range(2, 6):
        istr = '%d' % i
        wb = jnp.stack([p['lin%s_w' % istr][:, 0], p['lin%s_b' % istr]])
        part = _edge_pass(xp, sdp, ewp, wb)
        bn_s, bn_b = bn_fold(i)
        scal = make_scal(i, bn_s, bn_b)
        xp = _node_mlp16(xp, part[0, :N_NODES], part[1, :N_NODES],
                         p['nn%s_w1' % istr].T, p['nn%s_b1' % istr][None, :],
                         p['nn%s_w2' % istr].T, p['nn%s_b2' % istr][None, :],
                         scal)

    batch2d = pro_batch.reshape(N_NODES // 4000, 1, 4000)
    psum, pcnt = _pool(batch2d, xp)
    xp_out = _fc_pool(psum, pcnt, p['fc1_xp_w'].T, p['fc1_xp_b'][None, :])
    return (xc_rna, xp_out)
